# Initial kernel scaffold; baseline (speedup 1.0000x reference)
import jax
import jax.numpy as jnp
from jax.experimental import pallas as pl


def kernel(x, edge_attr, edge_index, node_graph_id, params):
    raise NotImplementedError("write your pallas kernel here")


# speedup vs baseline: 1.9100x; 1.9100x over previous
import jax
import jax.numpy as jnp
from jax.experimental import pallas as pl
from jax.experimental.pallas import tpu as pltpu

_N = 10000
_E = 320000
_G = 256
_NH = 336
_STEPS = 3

_RB = 256
_NPAD = 10240


def _dot(a, b):
    return jnp.dot(a, b, preferred_element_type=jnp.float32)


def _gru_body(agg_ref, h_ref, wxr_ref, whr_ref, wxz_ref, whz_ref, wxn_ref,
              whn_ref, br_ref, bz_ref, bn_ref, out_ref):
    agg = agg_ref[...]
    h = h_ref[...]
    r = jax.nn.sigmoid(_dot(agg, wxr_ref[...]) + _dot(h, whr_ref[...]) + br_ref[...])
    z = jax.nn.sigmoid(_dot(agg, wxz_ref[...]) + _dot(h, whz_ref[...]) + bz_ref[...])
    c = jnp.tanh(_dot(agg, wxn_ref[...]) + r * _dot(h, whn_ref[...]) + bn_ref[...])
    out_ref[...] = (1.0 - z) * c + z * h + h


def _gru_step(agg, h, bp):
    row_spec = pl.BlockSpec((_RB, _NH), lambda i: (i, 0))
    w_spec = pl.BlockSpec((_NH, _NH), lambda i: (0, 0))
    b_spec = pl.BlockSpec((1, _NH), lambda i: (0, 0))
    out = pl.pallas_call(
        _gru_body,
        grid=(_NPAD // _RB,),
        in_specs=[row_spec, row_spec] + [w_spec] * 6 + [b_spec] * 3,
        out_specs=row_spec,
        out_shape=jax.ShapeDtypeStruct((_NPAD, _NH), jnp.float32),
    )(agg, h, bp['Wxr'], bp['Whr'], bp['Wxz'], bp['Whz'], bp['Wxn'], bp['Whn'],
      bp['br'].reshape(1, _NH), bp['bz'].reshape(1, _NH), bp['bn'].reshape(1, _NH))
    return out


def kernel(x, edge_attr, edge_index, node_graph_id, params):
    p = params
    src = edge_index[0]
    dst = edge_index[1]
    gidd = node_graph_id[dst]

    ee = jax.nn.relu(edge_attr @ p['Wpe'] + p['bpe'])
    ee_g = jax.ops.segment_sum(ee, gidd, num_segments=_G)
    cmat = jnp.zeros((_G, _N), jnp.float32).at[gidd, src].add(1.0)

    ms = []
    for nm in ('gnn', 'scene', 'rgcn'):
        bp = p[nm]
        wm_top = bp['Wm'][:_NH]
        wm_bot = bp['Wm'][_NH:]
        h = jax.nn.relu(x @ bp['Wp'] + bp['bp'])
        eh = jax.nn.relu(edge_attr @ bp['We1'] + bp['be1'])
        ec = eh @ (bp['We2'] @ wm_bot) + (bp['be2'] @ wm_bot + bp['bm'])
        h = jnp.pad(h, ((0, _NPAD - _N), (0, 0)))
        for _ in range(_STEPS):
            hm = h @ wm_top
            msg = jax.nn.relu(hm[src] + ec)
            agg = jax.ops.segment_sum(msg, dst, num_segments=_NPAD)
            h = _gru_step(agg, h, bp)
        h = h[:_N]
        ms.append(jnp.concatenate([cmat @ h, ee_g], axis=1))

    f = jnp.stack(ms, axis=1)
    q = f @ p['Wq'] + p['bq']
    k = f @ p['Wk'] + p['bk']
    v = f @ p['Wv'] + p['bv']
    att = jax.nn.softmax(q @ jnp.swapaxes(k, -2, -1), axis=-1)
    comb = (att @ v).sum(axis=1)
    comb = jax.nn.softmax(comb, axis=1)
    h1 = jax.nn.relu(comb @ p['W1'] + p['b1'])
    emb = jax.nn.relu(h1 @ p['W2'] + p['b2'])
    out = emb @ p['W3'] + p['b3']
    logits = out.reshape(-1, 138, 1)
    proba = jax.nn.sigmoid(logits).squeeze(-1)
    return (proba, logits, emb)
